# baseline (speedup 1.0000x reference)
import functools

import jax
import jax.numpy as jnp
from jax import lax
from jax.experimental import pallas as pl
from jax.experimental.pallas import tpu as pltpu
from jax.experimental.pallas import tpu_sc as plsc

_NC = 2
_NS = 16
_NU = 4
_LN2 = 0.6931471805599453
_LOGP = (2.211703e-05, 0.99901044, -0.48915684, 0.28330433, -0.13011941,
         0.030102625)


def _threefry_bits(j):
    ks0 = jnp.uint32(0)
    ks1 = jnp.uint32(42)
    ks2 = jnp.uint32(0x1BD11BDA ^ 42)

    x0 = jnp.zeros_like(j) + ks0
    x1 = j + ks1

    rots = ((13, 15, 26, 6), (17, 29, 16, 24))
    ks = (ks0, ks1, ks2)
    for i in range(5):
        for r in rots[i % 2]:
            x0 = x0 + x1
            x1 = (x1 << r) | (x1 >> (32 - r))
            x1 = x1 ^ x0
        x0 = x0 + ks[(i + 1) % 3]
        x1 = x1 + ks[(i + 2) % 3] + jnp.uint32(i + 1)
    return x0 ^ x1


def _uniform_from_bits(bits):
    fb = (bits >> jnp.uint32(9)) | jnp.uint32(0x3F800000)
    return lax.bitcast_convert_type(fb, jnp.float32) - jnp.float32(1.0)


def _q_logfree(u):
    qs = jnp.float32(1.0 / 7.0)
    for k in (6, 5, 4, 3, 2, 1):
        qs = jnp.float32(1.0 / k) + u * qs
    qs = u * qs
    w = jnp.float32(1.0) - u
    wb = lax.bitcast_convert_type(w, jnp.uint32)
    e = (wb >> jnp.uint32(23)).astype(jnp.int32) - 127
    mant = lax.bitcast_convert_type(
        (wb & jnp.uint32(0x7FFFFF)) | jnp.uint32(0x3F800000), jnp.float32)
    f = mant - jnp.float32(1.0)
    poly = jnp.float32(_LOGP[5])
    for k in (4, 3, 2, 1, 0):
        poly = jnp.float32(_LOGP[k]) + f * poly
    q0 = jnp.float32(-_LN2) * e.astype(jnp.float32) - poly
    q1 = q0 + (jnp.float32(1.0) - w * jnp.exp(q0))
    return jnp.where(u < jnp.float32(0.125), qs, q1)


_NSL = 4


def _sc_body(v_total, ch, n_chunks, w4,
             logits_hbm, invt_hbm, a_hbm, q_hbm, c_hbm,
             xbuf, ibuf, bav, bqv, bcv):
    cc = lax.axis_index("c")
    ss = lax.axis_index("s")
    wid = ss * _NC + cc
    rb = wid // _NSL
    sl = wid % _NSL
    s0 = sl * w4
    iota = lax.iota(jnp.int32, 16)

    pltpu.sync_copy(invt_hbm.at[rb], ibuf)

    def initv(i, _):
        bav[pl.ds(i * 16, 16)] = jnp.zeros((16,), jnp.float32)
        bqv[pl.ds(i * 16, 16)] = jnp.ones((16,), jnp.float32)
        bcv[pl.ds(i * 16, 16)] = jnp.zeros((16,), jnp.int32)
        return 0

    lax.fori_loop(0, 32, initv, 0)

    def chunk_body(chk, _):
        pltpu.sync_copy(logits_hbm.at[rb, :, pl.ds(s0 + chk * ch, ch)], xbuf)
        for s in range(8):
            rowv = (rb * 8 + s) * v_total
            invt = ibuf[s, :]
            carry = []
            for k in range(_NU):
                carry.append(bav[pl.ds(s * 64 + k * 16, 16)])
                carry.append(bqv[pl.ds(s * 64 + k * 16, 16)])
                carry.append(bcv[pl.ds(s * 64 + k * 16, 16)])

            def group(g, carry, s=s, rowv=rowv, invt=invt, chk=chk):
                out = []
                base = s0 + chk * ch + g * (16 * _NU)
                for k in range(_NU):
                    ba, bq, bc = (carry[3 * k], carry[3 * k + 1],
                                  carry[3 * k + 2])
                    x16 = xbuf[s, pl.ds(g * (16 * _NU) + k * 16, 16)]
                    col = iota + (base + k * 16)
                    j = (rowv + col).astype(jnp.uint32)
                    u = _uniform_from_bits(_threefry_bits(j))
                    q = _q_logfree(u)
                    a = jnp.exp(x16 * invt)
                    upd = a * bq > ba * q
                    out.append(jnp.where(upd, a, ba))
                    out.append(jnp.where(upd, q, bq))
                    out.append(jnp.where(upd, col, bc))
                return tuple(out)

            res = lax.fori_loop(0, ch // (16 * _NU), group, tuple(carry))
            for k in range(_NU):
                bav[pl.ds(s * 64 + k * 16, 16)] = res[3 * k]
                bqv[pl.ds(s * 64 + k * 16, 16)] = res[3 * k + 1]
                bcv[pl.ds(s * 64 + k * 16, 16)] = res[3 * k + 2]
        return 0

    lax.fori_loop(0, n_chunks, chunk_body, 0)
    pltpu.sync_copy(bav, a_hbm.at[pl.ds(wid * 512, 512)])
    pltpu.sync_copy(bqv, q_hbm.at[pl.ds(wid * 512, 512)])
    pltpu.sync_copy(bcv, c_hbm.at[pl.ds(wid * 512, 512)])


def _sc_sampler(logits, invt16, v_sc, ch):
    b, v = logits.shape
    w4 = v_sc // _NSL
    assert v_sc % (_NSL * 128) == 0 and w4 % ch == 0 and ch % 128 == 0
    n_chunks = w4 // ch
    mesh = plsc.VectorSubcoreMesh(core_axis_name="c", subcore_axis_name="s",
                                  num_cores=_NC, num_subcores=_NS)
    body = functools.partial(_sc_body, v, ch, n_chunks, w4)
    n_out = 32 * 512
    f = pl.kernel(
        body,
        out_type=[
            jax.ShapeDtypeStruct((n_out,), jnp.float32),
            jax.ShapeDtypeStruct((n_out,), jnp.float32),
            jax.ShapeDtypeStruct((n_out,), jnp.int32),
        ],
        mesh=mesh,
        scratch_types=[
            pltpu.VMEM((8, ch), jnp.float32),
            pltpu.VMEM((8, 16), jnp.float32),
            pltpu.VMEM((512,), jnp.float32),
            pltpu.VMEM((512,), jnp.float32),
            pltpu.VMEM((512,), jnp.int32),
        ],
    )
    a, q, c = f(logits.reshape(8, b // 8, v), invt16.reshape(8, b // 8, 16))

    def rearr(x):
        return x.reshape(8, _NSL, 8, 64).transpose(0, 2, 1, 3).reshape(b, 256)

    return rearr(a), rearr(q), rearr(c)


_TCR = 8
_TLW = 256


def _tc_body(v_total, n_steps, chunk, blk0, logits_ref, invt_ref, val_ref,
             idx_ref, best_val, best_idx):
    g = pl.program_id(1)
    rg = pl.program_id(0)
    b = _TCR

    invt = invt_ref[...]
    lane = lax.broadcasted_iota(jnp.int32, (b, _TLW), 1)
    rowv = ((lax.broadcasted_iota(jnp.int32, (b, _TLW), 0) + rg * b)
            * v_total)
    cbase = (g + blk0) * chunk

    def tile(i, carry):
        bv, bi = carry
        off = pl.multiple_of(i * _TLW, _TLW)
        x = logits_ref[:, pl.ds(off, _TLW)]
        col = lane + (cbase + off)
        j = (rowv + col).astype(jnp.uint32)
        u = _uniform_from_bits(_threefry_bits(j))
        q = -jnp.log1p(-u)
        s = x * invt - jnp.log(q)
        s = jnp.where(col < v_total, s, -jnp.inf)
        upd = s > bv
        return jnp.where(upd, s, bv), jnp.where(upd, col, bi)

    bv0 = jnp.full((b, _TLW), -jnp.inf, jnp.float32)
    bi0 = jnp.zeros((b, _TLW), jnp.int32)
    bv, bi = lax.fori_loop(0, chunk // _TLW, tile, (bv0, bi0))

    m = jnp.max(bv, axis=1, keepdims=True)
    idx = jnp.min(jnp.where(bv == m, bi, v_total), axis=1, keepdims=True)

    @pl.when(g == 0)
    def _init():
        best_val[...] = jnp.full_like(best_val, -jnp.inf)
        best_idx[...] = jnp.zeros_like(best_idx)

    better = m > best_val[...]
    best_idx[...] = jnp.where(better, idx, best_idx[...])
    best_val[...] = jnp.where(better, m, best_val[...])

    @pl.when(g == n_steps - 1)
    def _done():
        val_ref[...] = best_val[...]
        idx_ref[...] = best_idx[...]


def _tc_partial(logits, invt, v_total, v_sc, chunk):
    b, v = logits.shape
    assert v_sc % chunk == 0 and chunk % _TLW == 0
    blk0 = v_sc // chunk
    n_steps = pl.cdiv(v_total - v_sc, chunk)
    body = functools.partial(_tc_body, v_total, n_steps, chunk, blk0)
    return pl.pallas_call(
        body,
        grid=(b // _TCR, n_steps),
        in_specs=[
            pl.BlockSpec((_TCR, chunk), lambda rg, g, blk0=blk0: (rg, g + blk0)),
            pl.BlockSpec((_TCR, 1), lambda rg, g: (rg, 0)),
        ],
        out_specs=[
            pl.BlockSpec((_TCR, 1), lambda rg, g: (rg, 0)),
            pl.BlockSpec((_TCR, 1), lambda rg, g: (rg, 0)),
        ],
        out_shape=[
            jax.ShapeDtypeStruct((b, 1), jnp.float32),
            jax.ShapeDtypeStruct((b, 1), jnp.int32),
        ],
        scratch_shapes=[
            pltpu.VMEM((_TCR, 1), jnp.float32),
            pltpu.VMEM((_TCR, 1), jnp.int32),
        ],
    )(logits, invt)


def _merge_body(v_total, a_ref, q_ref, c_ref, tv_ref, ti_ref, out_ref):
    s = jnp.log(a_ref[...]) - jnp.log(q_ref[...])
    m = jnp.max(s, axis=1, keepdims=True)
    idx = jnp.min(jnp.where(s == m, c_ref[...], v_total), axis=1,
                  keepdims=True)
    pick_sc = m >= tv_ref[...]
    out_ref[...] = jnp.where(pick_sc, idx, ti_ref[...])


def _merge(v_total, a, q, c, tv, ti):
    b = a.shape[0]
    return pl.pallas_call(
        functools.partial(_merge_body, v_total),
        out_shape=jax.ShapeDtypeStruct((b, 1), jnp.int32),
    )(a, q, c, tv, ti)


def _pick_sc_chunk(width, cap=12288):
    for cand in range(cap - cap % 128, 127, -128):
        if width % cand == 0:
            return cand
    return None


def kernel(logits, temperatures):
    b, v = logits.shape
    logits = logits.astype(jnp.float32)
    invt = (1.0 / temperatures.astype(jnp.float32)).reshape(b, 1)

    chunk = 16384
    align = _NSL * chunk
    v_sc = ((v * 84) // 256) // align * align
    ch = _pick_sc_chunk(v_sc // _NSL)

    invt16 = jnp.broadcast_to(invt, (b, 16))
    a, q, c = _sc_sampler(logits, invt16, v_sc, ch)
    tv, ti = _tc_partial(logits, invt, v, v_sc, chunk)
    out = _merge(v, a, q, c, tv, ti)
    return out.reshape(b)


# speedup vs baseline: 1.5637x; 1.5637x over previous
import functools

import jax
import jax.numpy as jnp
from jax import lax
from jax.experimental import pallas as pl
from jax.experimental.pallas import tpu as pltpu
from jax.experimental.pallas import tpu_sc as plsc

_NC = 2
_NS = 16
_NU = 4
_LN2 = 0.6931471805599453
_LOGP = (2.211703e-05, 0.99901044, -0.48915684, 0.28330433, -0.13011941,
         0.030102625)


def _threefry_bits(j):
    ks0 = jnp.uint32(0)
    ks1 = jnp.uint32(42)
    ks2 = jnp.uint32(0x1BD11BDA ^ 42)

    x0 = jnp.zeros_like(j) + ks0
    x1 = j + ks1

    rots = ((13, 15, 26, 6), (17, 29, 16, 24))
    ks = (ks0, ks1, ks2)
    for i in range(5):
        for r in rots[i % 2]:
            x0 = x0 + x1
            x1 = (x1 << r) | (x1 >> (32 - r))
            x1 = x1 ^ x0
        x0 = x0 + ks[(i + 1) % 3]
        x1 = x1 + ks[(i + 2) % 3] + jnp.uint32(i + 1)
    return x0 ^ x1


def _uniform_from_bits(bits):
    fb = (bits >> jnp.uint32(9)) | jnp.uint32(0x3F800000)
    return lax.bitcast_convert_type(fb, jnp.float32) - jnp.float32(1.0)


def _q_logfree(u):
    qs = jnp.float32(1.0 / 7.0)
    for k in (6, 5, 4, 3, 2, 1):
        qs = jnp.float32(1.0 / k) + u * qs
    qs = u * qs
    w = jnp.float32(1.0) - u
    wb = lax.bitcast_convert_type(w, jnp.uint32)
    e = (wb >> jnp.uint32(23)).astype(jnp.int32) - 127
    mant = lax.bitcast_convert_type(
        (wb & jnp.uint32(0x7FFFFF)) | jnp.uint32(0x3F800000), jnp.float32)
    f = mant - jnp.float32(1.0)
    poly = jnp.float32(_LOGP[5])
    for k in (4, 3, 2, 1, 0):
        poly = jnp.float32(_LOGP[k]) + f * poly
    q0 = jnp.float32(-_LN2) * e.astype(jnp.float32) - poly
    q1 = q0 + (jnp.float32(1.0) - w * jnp.exp(q0))
    return jnp.where(u < jnp.float32(0.125), qs, q1)


_NSL = 4


def _sc_body(v_total, ch, n_chunks, w4,
             logits_hbm, invt_hbm, a_hbm, q_hbm, c_hbm,
             xbuf, ibuf, bav, bqv, bcv):
    cc = lax.axis_index("c")
    ss = lax.axis_index("s")
    wid = ss * _NC + cc
    rb = wid // _NSL
    sl = wid % _NSL
    s0 = sl * w4
    iota = lax.iota(jnp.int32, 16)

    pltpu.sync_copy(invt_hbm.at[rb], ibuf)

    def initv(i, _):
        bav[pl.ds(i * 16, 16)] = jnp.zeros((16,), jnp.float32)
        bqv[pl.ds(i * 16, 16)] = jnp.ones((16,), jnp.float32)
        bcv[pl.ds(i * 16, 16)] = jnp.zeros((16,), jnp.int32)
        return 0

    lax.fori_loop(0, 32, initv, 0)

    def chunk_body(chk, _):
        pltpu.sync_copy(logits_hbm.at[rb, :, pl.ds(s0 + chk * ch, ch)], xbuf)
        for s in range(8):
            rowv = (rb * 8 + s) * v_total
            invt = ibuf[s, :]
            carry = []
            for k in range(_NU):
                carry.append(bav[pl.ds(s * 64 + k * 16, 16)])
                carry.append(bqv[pl.ds(s * 64 + k * 16, 16)])
                carry.append(bcv[pl.ds(s * 64 + k * 16, 16)])

            def group(g, carry, s=s, rowv=rowv, invt=invt, chk=chk):
                out = []
                base = s0 + chk * ch + g * (16 * _NU)
                for k in range(_NU):
                    ba, bq, bc = (carry[3 * k], carry[3 * k + 1],
                                  carry[3 * k + 2])
                    x16 = xbuf[s, pl.ds(g * (16 * _NU) + k * 16, 16)]
                    col = iota + (base + k * 16)
                    j = (rowv + col).astype(jnp.uint32)
                    u = _uniform_from_bits(_threefry_bits(j))
                    q = _q_logfree(u)
                    a = jnp.exp(x16 * invt)
                    upd = a * bq > ba * q
                    out.append(jnp.where(upd, a, ba))
                    out.append(jnp.where(upd, q, bq))
                    out.append(jnp.where(upd, col, bc))
                return tuple(out)

            res = lax.fori_loop(0, ch // (16 * _NU), group, tuple(carry))
            for k in range(_NU):
                bav[pl.ds(s * 64 + k * 16, 16)] = res[3 * k]
                bqv[pl.ds(s * 64 + k * 16, 16)] = res[3 * k + 1]
                bcv[pl.ds(s * 64 + k * 16, 16)] = res[3 * k + 2]
        return 0

    lax.fori_loop(0, n_chunks, chunk_body, 0)
    pltpu.sync_copy(bav, a_hbm.at[pl.ds(wid * 512, 512)])
    pltpu.sync_copy(bqv, q_hbm.at[pl.ds(wid * 512, 512)])
    pltpu.sync_copy(bcv, c_hbm.at[pl.ds(wid * 512, 512)])


def _sc_sampler(logits, invt16, v_sc, ch):
    b, v = logits.shape
    w4 = v_sc // _NSL
    assert v_sc % (_NSL * 128) == 0 and w4 % ch == 0 and ch % 128 == 0
    n_chunks = w4 // ch
    mesh = plsc.VectorSubcoreMesh(core_axis_name="c", subcore_axis_name="s",
                                  num_cores=_NC, num_subcores=_NS)
    body = functools.partial(_sc_body, v, ch, n_chunks, w4)
    n_out = 32 * 512
    f = pl.kernel(
        body,
        out_type=[
            jax.ShapeDtypeStruct((n_out,), jnp.float32),
            jax.ShapeDtypeStruct((n_out,), jnp.float32),
            jax.ShapeDtypeStruct((n_out,), jnp.int32),
        ],
        mesh=mesh,
        scratch_types=[
            pltpu.VMEM((8, ch), jnp.float32),
            pltpu.VMEM((8, 16), jnp.float32),
            pltpu.VMEM((512,), jnp.float32),
            pltpu.VMEM((512,), jnp.float32),
            pltpu.VMEM((512,), jnp.int32),
        ],
    )
    a, q, c = f(logits.reshape(8, b // 8, v), invt16.reshape(8, b // 8, 16))

    def rearr(x):
        return x.reshape(8, _NSL, 8, 64).transpose(0, 2, 1, 3).reshape(b, 256)

    return rearr(a), rearr(q), rearr(c)


def _tc_body(v_total, n_steps, chunk, blk0, logits_ref, invt_ref, val_ref,
             idx_ref, best_val, best_idx):
    g = pl.program_id(0)
    b = logits_ref.shape[0]

    x = logits_ref[...]
    col = lax.broadcasted_iota(jnp.int32, (b, chunk), 1) + (g + blk0) * chunk
    row = lax.broadcasted_iota(jnp.int32, (b, chunk), 0)
    j = (row * v_total + col).astype(jnp.uint32)

    u = _uniform_from_bits(_threefry_bits(j))
    q = -jnp.log1p(-u)
    s = x * invt_ref[...] - jnp.log(q)
    s = jnp.where(col < v_total, s, -jnp.inf)

    m = jnp.max(s, axis=1, keepdims=True)
    idx = jnp.min(jnp.where(s == m, col, v_total), axis=1, keepdims=True)

    @pl.when(g == 0)
    def _init():
        best_val[...] = jnp.full_like(best_val, -jnp.inf)
        best_idx[...] = jnp.zeros_like(best_idx)

    better = m > best_val[...]
    best_idx[...] = jnp.where(better, idx, best_idx[...])
    best_val[...] = jnp.where(better, m, best_val[...])

    @pl.when(g == n_steps - 1)
    def _done():
        val_ref[...] = best_val[...]
        idx_ref[...] = best_idx[...]


def _tc_partial(logits, invt, v_total, v_sc, chunk):
    b, v = logits.shape
    assert v_sc % chunk == 0
    blk0 = v_sc // chunk
    n_steps = pl.cdiv(v_total - v_sc, chunk)
    body = functools.partial(_tc_body, v_total, n_steps, chunk, blk0)
    return pl.pallas_call(
        body,
        grid=(n_steps,),
        in_specs=[
            pl.BlockSpec((b, chunk), lambda g, blk0=blk0: (0, g + blk0)),
            pl.BlockSpec((b, 1), lambda g: (0, 0)),
        ],
        out_specs=[
            pl.BlockSpec((b, 1), lambda g: (0, 0)),
            pl.BlockSpec((b, 1), lambda g: (0, 0)),
        ],
        out_shape=[
            jax.ShapeDtypeStruct((b, 1), jnp.float32),
            jax.ShapeDtypeStruct((b, 1), jnp.int32),
        ],
        scratch_shapes=[
            pltpu.VMEM((b, 1), jnp.float32),
            pltpu.VMEM((b, 1), jnp.int32),
        ],
    )(logits, invt)


def _merge_body(v_total, a_ref, q_ref, c_ref, tv_ref, ti_ref, out_ref):
    s = jnp.log(a_ref[...]) - jnp.log(q_ref[...])
    m = jnp.max(s, axis=1, keepdims=True)
    idx = jnp.min(jnp.where(s == m, c_ref[...], v_total), axis=1,
                  keepdims=True)
    pick_sc = m >= tv_ref[...]
    out_ref[...] = jnp.where(pick_sc, idx, ti_ref[...])


def _merge(v_total, a, q, c, tv, ti):
    b = a.shape[0]
    return pl.pallas_call(
        functools.partial(_merge_body, v_total),
        out_shape=jax.ShapeDtypeStruct((b, 1), jnp.int32),
    )(a, q, c, tv, ti)


def _pick_sc_chunk(width, cap=12288):
    for cand in range(cap - cap % 128, 127, -128):
        if width % cand == 0:
            return cand
    return None


def kernel(logits, temperatures):
    b, v = logits.shape
    logits = logits.astype(jnp.float32)
    invt = (1.0 / temperatures.astype(jnp.float32)).reshape(b, 1)

    chunk = 32768
    align = _NSL * chunk // 2
    v_sc = ((v * 84) // 256) // align * align
    ch = _pick_sc_chunk(v_sc // _NSL)

    invt16 = jnp.broadcast_to(invt, (b, 16))
    a, q, c = _sc_sampler(logits, invt16, v_sc, ch)
    tv, ti = _tc_partial(logits, invt, v, v_sc, chunk)
    out = _merge(v, a, q, c, tv, ti)
    return out.reshape(b)


# speedup vs baseline: 1.8691x; 1.1953x over previous
import functools

import jax
import jax.numpy as jnp
from jax import lax
from jax.experimental import pallas as pl
from jax.experimental.pallas import tpu as pltpu
from jax.experimental.pallas import tpu_sc as plsc

_NC = 2
_NS = 16
_NU = 4
_LN2 = 0.6931471805599453
_LOGP = (2.211703e-05, 0.99901044, -0.48915684, 0.28330433, -0.13011941,
         0.030102625)


def _threefry_bits(j):
    ks0 = jnp.uint32(0)
    ks1 = jnp.uint32(42)
    ks2 = jnp.uint32(0x1BD11BDA ^ 42)

    x0 = jnp.zeros_like(j) + ks0
    x1 = j + ks1

    rots = ((13, 15, 26, 6), (17, 29, 16, 24))
    ks = (ks0, ks1, ks2)
    for i in range(5):
        for r in rots[i % 2]:
            x0 = x0 + x1
            x1 = (x1 << r) | (x1 >> (32 - r))
            x1 = x1 ^ x0
        x0 = x0 + ks[(i + 1) % 3]
        x1 = x1 + ks[(i + 2) % 3] + jnp.uint32(i + 1)
    return x0 ^ x1


def _uniform_from_bits(bits):
    fb = (bits >> jnp.uint32(9)) | jnp.uint32(0x3F800000)
    return lax.bitcast_convert_type(fb, jnp.float32) - jnp.float32(1.0)


def _q_logfree(u):
    qs = jnp.float32(1.0 / 7.0)
    for k in (6, 5, 4, 3, 2, 1):
        qs = jnp.float32(1.0 / k) + u * qs
    qs = u * qs
    w = jnp.float32(1.0) - u
    wb = lax.bitcast_convert_type(w, jnp.uint32)
    e = (wb >> jnp.uint32(23)).astype(jnp.int32) - 127
    mant = lax.bitcast_convert_type(
        (wb & jnp.uint32(0x7FFFFF)) | jnp.uint32(0x3F800000), jnp.float32)
    f = mant - jnp.float32(1.0)
    poly = jnp.float32(_LOGP[5])
    for k in (4, 3, 2, 1, 0):
        poly = jnp.float32(_LOGP[k]) + f * poly
    q0 = jnp.float32(-_LN2) * e.astype(jnp.float32) - poly
    q1 = q0 + (jnp.float32(1.0) - w * jnp.exp(q0))
    return jnp.where(u < jnp.float32(0.125), qs, q1)


_NSL = 4


def _sc_body(v_total, ch, n_chunks, w4,
             logits_hbm, invt_hbm, a_hbm, q_hbm, c_hbm,
             xbuf, ibuf, bav, bqv, bcv):
    cc = lax.axis_index("c")
    ss = lax.axis_index("s")
    wid = ss * _NC + cc
    rb = wid // _NSL
    sl = wid % _NSL
    s0 = sl * w4
    iota = lax.iota(jnp.int32, 16)

    pltpu.sync_copy(invt_hbm.at[rb], ibuf)

    def initv(i, _):
        bav[pl.ds(i * 16, 16)] = jnp.zeros((16,), jnp.float32)
        bqv[pl.ds(i * 16, 16)] = jnp.ones((16,), jnp.float32)
        bcv[pl.ds(i * 16, 16)] = jnp.zeros((16,), jnp.int32)
        return 0

    lax.fori_loop(0, 32, initv, 0)

    def chunk_body(chk, _):
        pltpu.sync_copy(logits_hbm.at[rb, :, pl.ds(s0 + chk * ch, ch)], xbuf)
        for s in range(8):
            rowv = (rb * 8 + s) * v_total
            invt = ibuf[s, :]
            carry = []
            for k in range(_NU):
                carry.append(bav[pl.ds(s * 64 + k * 16, 16)])
                carry.append(bqv[pl.ds(s * 64 + k * 16, 16)])
                carry.append(bcv[pl.ds(s * 64 + k * 16, 16)])

            def group(g, carry, s=s, rowv=rowv, invt=invt, chk=chk):
                out = []
                base = s0 + chk * ch + g * (16 * _NU)
                for k in range(_NU):
                    ba, bq, bc = (carry[3 * k], carry[3 * k + 1],
                                  carry[3 * k + 2])
                    x16 = xbuf[s, pl.ds(g * (16 * _NU) + k * 16, 16)]
                    col = iota + (base + k * 16)
                    j = (rowv + col).astype(jnp.uint32)
                    u = _uniform_from_bits(_threefry_bits(j))
                    q = _q_logfree(u)
                    a = jnp.exp(x16 * invt)
                    upd = a * bq > ba * q
                    out.append(jnp.where(upd, a, ba))
                    out.append(jnp.where(upd, q, bq))
                    out.append(jnp.where(upd, col, bc))
                return tuple(out)

            res = lax.fori_loop(0, ch // (16 * _NU), group, tuple(carry))
            for k in range(_NU):
                bav[pl.ds(s * 64 + k * 16, 16)] = res[3 * k]
                bqv[pl.ds(s * 64 + k * 16, 16)] = res[3 * k + 1]
                bcv[pl.ds(s * 64 + k * 16, 16)] = res[3 * k + 2]
        return 0

    lax.fori_loop(0, n_chunks, chunk_body, 0)
    pltpu.sync_copy(bav, a_hbm.at[pl.ds(wid * 512, 512)])
    pltpu.sync_copy(bqv, q_hbm.at[pl.ds(wid * 512, 512)])
    pltpu.sync_copy(bcv, c_hbm.at[pl.ds(wid * 512, 512)])


def _sc_sampler(logits, invt16, v_sc, ch):
    b, v = logits.shape
    w4 = v_sc // _NSL
    assert v_sc % (_NSL * 128) == 0 and w4 % ch == 0 and ch % 128 == 0
    n_chunks = w4 // ch
    mesh = plsc.VectorSubcoreMesh(core_axis_name="c", subcore_axis_name="s",
                                  num_cores=_NC, num_subcores=_NS)
    body = functools.partial(_sc_body, v, ch, n_chunks, w4)
    n_out = 32 * 512
    f = pl.kernel(
        body,
        out_type=[
            jax.ShapeDtypeStruct((n_out,), jnp.float32),
            jax.ShapeDtypeStruct((n_out,), jnp.float32),
            jax.ShapeDtypeStruct((n_out,), jnp.int32),
        ],
        mesh=mesh,
        scratch_types=[
            pltpu.VMEM((8, ch), jnp.float32),
            pltpu.VMEM((8, 16), jnp.float32),
            pltpu.VMEM((512,), jnp.float32),
            pltpu.VMEM((512,), jnp.float32),
            pltpu.VMEM((512,), jnp.int32),
        ],
    )
    a, q, c = f(logits.reshape(8, b // 8, v), invt16.reshape(8, b // 8, 16))

    def rearr(x):
        return x.reshape(8, _NSL, 8, 64).transpose(0, 2, 1, 3).reshape(b, 256)

    return rearr(a), rearr(q), rearr(c)


def _tc_body(v_total, n_steps, chunk, blk0, logits_ref, invt_ref, val_ref,
             idx_ref, best_val, best_idx):
    g = pl.program_id(0)
    b = logits_ref.shape[0]

    x = logits_ref[...]
    col = lax.broadcasted_iota(jnp.int32, (b, chunk), 1) + (g + blk0) * chunk
    row = lax.broadcasted_iota(jnp.int32, (b, chunk), 0)
    j = (row * v_total + col).astype(jnp.uint32)

    u = _uniform_from_bits(_threefry_bits(j))
    q = -jnp.log1p(-u)
    s = x * invt_ref[...] - jnp.log(q)
    s = jnp.where(col < v_total, s, -jnp.inf)

    m = jnp.max(s, axis=1, keepdims=True)
    idx = jnp.min(jnp.where(s == m, col, v_total), axis=1, keepdims=True)

    @pl.when(g == 0)
    def _init():
        best_val[...] = jnp.full_like(best_val, -jnp.inf)
        best_idx[...] = jnp.zeros_like(best_idx)

    better = m > best_val[...]
    best_idx[...] = jnp.where(better, idx, best_idx[...])
    best_val[...] = jnp.where(better, m, best_val[...])

    @pl.when(g == n_steps - 1)
    def _done():
        val_ref[...] = best_val[...]
        idx_ref[...] = best_idx[...]


def _tc_partial(logits, invt, v_total, v_sc, chunk):
    b, v = logits.shape
    assert v_sc % chunk == 0
    blk0 = v_sc // chunk
    n_steps = pl.cdiv(v_total - v_sc, chunk)
    body = functools.partial(_tc_body, v_total, n_steps, chunk, blk0)
    return pl.pallas_call(
        body,
        grid=(n_steps,),
        in_specs=[
            pl.BlockSpec((b, chunk), lambda g, blk0=blk0: (0, g + blk0)),
            pl.BlockSpec((b, 1), lambda g: (0, 0)),
        ],
        out_specs=[
            pl.BlockSpec((b, 1), lambda g: (0, 0)),
            pl.BlockSpec((b, 1), lambda g: (0, 0)),
        ],
        out_shape=[
            jax.ShapeDtypeStruct((b, 1), jnp.float32),
            jax.ShapeDtypeStruct((b, 1), jnp.int32),
        ],
        scratch_shapes=[
            pltpu.VMEM((b, 1), jnp.float32),
            pltpu.VMEM((b, 1), jnp.int32),
        ],
    )(logits, invt)


def _merge_body(v_total, a_ref, q_ref, c_ref, tv_ref, ti_ref, out_ref):
    s = jnp.log(a_ref[...]) - jnp.log(q_ref[...])
    m = jnp.max(s, axis=1, keepdims=True)
    idx = jnp.min(jnp.where(s == m, c_ref[...], v_total), axis=1,
                  keepdims=True)
    pick_sc = m >= tv_ref[...]
    out_ref[...] = jnp.where(pick_sc, idx, ti_ref[...])


def _merge(v_total, a, q, c, tv, ti):
    b = a.shape[0]
    return pl.pallas_call(
        functools.partial(_merge_body, v_total),
        out_shape=jax.ShapeDtypeStruct((b, 1), jnp.int32),
    )(a, q, c, tv, ti)


def _pick_sc_chunk(width, cap=12288):
    for cand in range(cap - cap % 128, 127, -128):
        if width % cand == 0:
            return cand
    return None


def kernel(logits, temperatures):
    b, v = logits.shape
    logits = logits.astype(jnp.float32)
    invt = (1.0 / temperatures.astype(jnp.float32)).reshape(b, 1)

    chunk = 16384
    align = _NSL * chunk
    v_sc = ((v * 84) // 256) // align * align
    ch = _pick_sc_chunk(v_sc // _NSL)

    invt16 = jnp.broadcast_to(invt, (b, 16))
    a, q, c = _sc_sampler(logits, invt16, v_sc, ch)
    tv, ti = _tc_partial(logits, invt, v, v_sc, chunk)
    out = _merge(v, a, q, c, tv, ti)
    return out.reshape(b)
